# Initial kernel scaffold; baseline (speedup 1.0000x reference)
import jax
import jax.numpy as jnp
from jax.experimental import pallas as pl


def kernel(x, W_in, W_proc):
    raise NotImplementedError("write your pallas kernel here")


# speedup vs baseline: 1.4511x; 1.4511x over previous
import jax
import jax.numpy as jnp
from jax import lax
from jax.experimental import pallas as pl
from jax.experimental.pallas import tpu as pltpu

D_MODEL_K = 4096
N_IN = 8
N_PROC = 32
TOPK = 3
BLK = 512


def _router_body(x_ref, win_ref, wproc_ref, w_out_ref, idx_out_ref):
    xb = x_ref[...]
    s_in = jax.lax.dot_general(
        xb, win_ref[...], (((1,), (0,)), ((), ())),
        preferred_element_type=jnp.float32)
    s_pr = jax.lax.dot_general(
        xb, wproc_ref[...], (((1,), (0,)), ((), ())),
        preferred_element_type=jnp.float32)

    m = jnp.max(s_in, axis=-1, keepdims=True)
    e = jnp.exp(s_in - m)
    w_out_ref[...] = e / jnp.sum(e, axis=-1, keepdims=True)

    iota = lax.broadcasted_iota(jnp.int32, s_pr.shape, 1)
    s = s_pr
    cols = []
    for _ in range(TOPK):
        mx = jnp.max(s, axis=-1, keepdims=True)
        idx = jnp.min(jnp.where(s == mx, iota, N_PROC), axis=-1, keepdims=True)
        cols.append(idx)
        s = jnp.where(iota == idx, -jnp.inf, s)
    idx_out_ref[...] = jnp.concatenate(cols, axis=1)


def kernel(x, W_in, W_proc):
    B, S, D = x.shape
    T = B * S
    x2 = x.reshape(T, D)
    grid = (T // BLK,)
    weights, indices = pl.pallas_call(
        _router_body,
        grid=grid,
        in_specs=[
            pl.BlockSpec((BLK, D), lambda i: (i, 0)),
            pl.BlockSpec((D, N_IN), lambda i: (0, 0)),
            pl.BlockSpec((D, N_PROC), lambda i: (0, 0)),
        ],
        out_specs=[
            pl.BlockSpec((BLK, N_IN), lambda i: (i, 0)),
            pl.BlockSpec((BLK, TOPK), lambda i: (i, 0)),
        ],
        out_shape=[
            jax.ShapeDtypeStruct((T, N_IN), jnp.float32),
            jax.ShapeDtypeStruct((T, TOPK), jnp.int32),
        ],
        compiler_params=pltpu.CompilerParams(
            dimension_semantics=("arbitrary",)),
    )(x2, W_in.T, W_proc.T)
    return (indices.reshape(B, S, TOPK), weights.reshape(B, S, N_IN))


# speedup vs baseline: 1.5143x; 1.0436x over previous
import jax
import jax.numpy as jnp
from jax import lax
from jax.experimental import pallas as pl
from jax.experimental.pallas import tpu as pltpu

D_MODEL_K = 4096
N_IN = 8
N_PROC = 32
TOPK = 3
BLK = 512


def _router_body(x_ref, w_ref, w_out_ref, idx_out_ref):
    xb = x_ref[...]
    s = jax.lax.dot_general(
        xb, w_ref[...], (((1,), (0,)), ((), ())),
        preferred_element_type=jnp.float32)
    s_in = s[:, :N_IN]
    s_pr = s[:, N_IN:]

    m = jnp.max(s_in, axis=-1, keepdims=True)
    e = jnp.exp(s_in - m)
    w_out_ref[...] = e / jnp.sum(e, axis=-1, keepdims=True)

    iota = lax.broadcasted_iota(jnp.int32, s_pr.shape, 1)
    s = s_pr
    cols = []
    for _ in range(TOPK):
        mx = jnp.max(s, axis=-1, keepdims=True)
        idx = jnp.min(jnp.where(s == mx, iota, N_PROC), axis=-1, keepdims=True)
        cols.append(idx)
        s = jnp.where(iota == idx, -jnp.inf, s)
    idx_out_ref[...] = jnp.concatenate(cols, axis=1)


def kernel(x, W_in, W_proc):
    B, S, D = x.shape
    T = B * S
    x2 = x.reshape(T, D)
    grid = (T // BLK,)
    weights, indices = pl.pallas_call(
        _router_body,
        grid=grid,
        in_specs=[
            pl.BlockSpec((BLK, D), lambda i: (i, 0)),
            pl.BlockSpec((D, N_IN + N_PROC), lambda i: (0, 0)),
        ],
        out_specs=[
            pl.BlockSpec((BLK, N_IN), lambda i: (i, 0)),
            pl.BlockSpec((BLK, TOPK), lambda i: (i, 0)),
        ],
        out_shape=[
            jax.ShapeDtypeStruct((T, N_IN), jnp.float32),
            jax.ShapeDtypeStruct((T, TOPK), jnp.int32),
        ],
        compiler_params=pltpu.CompilerParams(
            dimension_semantics=("arbitrary",)),
    )(x2, jnp.concatenate([W_in, W_proc], axis=0).T)
    return (indices.reshape(B, S, TOPK), weights.reshape(B, S, N_IN))


# speedup vs baseline: 1.6708x; 1.1034x over previous
import jax
import jax.numpy as jnp
from jax import lax
from jax.experimental import pallas as pl
from jax.experimental.pallas import tpu as pltpu

D_MODEL_K = 4096
N_IN = 8
N_PROC = 32
TOPK = 3
BLK = 1024


def _router_body(x_ref, w_ref, w_out_ref, idx_out_ref):
    xb = x_ref[...]
    s = jax.lax.dot_general(
        xb, w_ref[...], (((1,), (0,)), ((), ())),
        preferred_element_type=jnp.float32)
    s_in = s[:, :N_IN]
    s_pr = s[:, N_IN:]

    m = jnp.max(s_in, axis=-1, keepdims=True)
    e = jnp.exp(s_in - m)
    w_out_ref[...] = e / jnp.sum(e, axis=-1, keepdims=True)

    iota = lax.broadcasted_iota(jnp.int32, s_pr.shape, 1)
    s = s_pr
    cols = []
    for _ in range(TOPK):
        mx = jnp.max(s, axis=-1, keepdims=True)
        idx = jnp.min(jnp.where(s == mx, iota, N_PROC), axis=-1, keepdims=True)
        cols.append(idx)
        s = jnp.where(iota == idx, -jnp.inf, s)
    idx_out_ref[...] = jnp.concatenate(cols, axis=1)


def kernel(x, W_in, W_proc):
    B, S, D = x.shape
    T = B * S
    x2 = x.reshape(T, D)
    grid = (T // BLK,)
    weights, indices = pl.pallas_call(
        _router_body,
        grid=grid,
        in_specs=[
            pl.BlockSpec((BLK, D), lambda i: (i, 0)),
            pl.BlockSpec((D, N_IN + N_PROC), lambda i: (0, 0)),
        ],
        out_specs=[
            pl.BlockSpec((BLK, N_IN), lambda i: (i, 0)),
            pl.BlockSpec((BLK, TOPK), lambda i: (i, 0)),
        ],
        out_shape=[
            jax.ShapeDtypeStruct((T, N_IN), jnp.float32),
            jax.ShapeDtypeStruct((T, TOPK), jnp.int32),
        ],
        compiler_params=pltpu.CompilerParams(
            dimension_semantics=("arbitrary",)),
    )(x2, jnp.concatenate([W_in, W_proc], axis=0).T)
    return (indices.reshape(B, S, TOPK), weights.reshape(B, S, N_IN))
